# Initial kernel scaffold; baseline (speedup 1.0000x reference)
import jax
import jax.numpy as jnp
from jax.experimental import pallas as pl


def kernel(hidden_states, gate_w, ws, w2s):
    raise NotImplementedError("write your pallas kernel here")


# speedup vs baseline: 1.1478x; 1.1478x over previous
import jax
import jax.numpy as jnp
from jax.experimental import pallas as pl
from jax.experimental.pallas import tpu as pltpu

_E = 8
_H = 2048
_I = 2048
_T = 64
_JITTER = 0.01
_BI = 512
_NB = _I // _BI


def _sparsemixer_routing(scores):
    lanes = jax.lax.broadcasted_iota(jnp.int32, scores.shape, 1)
    neg_inf = jnp.float32(-jnp.inf)
    mlt = jnp.max(scores, axis=-1, keepdims=True)
    eq1 = scores == mlt
    ind1 = jnp.min(jnp.where(eq1, lanes, _E), axis=-1, keepdims=True)
    oh1 = lanes == ind1
    factor = jnp.maximum(jnp.abs(scores), mlt)
    mask1 = (mlt - scores) / factor > 2.0 * _JITTER
    mg1 = jnp.where(mask1, neg_inf, scores)
    sm1 = jax.nn.softmax(mg1, axis=-1)
    m1 = jnp.sum(jnp.where(oh1, sm1, 0.0), axis=-1, keepdims=True)
    masked_scores = jnp.where(oh1, neg_inf, scores)
    mlt2 = jnp.max(masked_scores, axis=-1, keepdims=True)
    eq2 = masked_scores == mlt2
    ind2 = jnp.min(jnp.where(eq2, lanes, _E), axis=-1, keepdims=True)
    oh2 = lanes == ind2
    factor2 = jnp.maximum(jnp.abs(scores), mlt2)
    mask2 = (mlt2 - scores) / factor2 > 2.0 * _JITTER
    mg2 = jnp.where(mask2, neg_inf, masked_scores)
    sm2 = jax.nn.softmax(mg2, axis=-1)
    m2 = jnp.sum(jnp.where(oh2, sm2, 0.0), axis=-1, keepdims=True)
    return jnp.where(oh1, m1, 0.0) + jnp.where(oh2, m2, 0.0)


def _moe_body(x_ref, gate_ref, w1_ref, w3_ref, w2_ref, out_ref, rt_ref):
    e = pl.program_id(0)
    i = pl.program_id(1)

    @pl.when((e == 0) & (i == 0))
    def _init():
        logits = jax.lax.dot_general(
            x_ref[...], gate_ref[...], (((1,), (1,)), ((), ())),
            preferred_element_type=jnp.float32)
        rt_ref[...] = _sparsemixer_routing(logits)
        out_ref[...] = jnp.zeros_like(out_ref)

    x = x_ref[...]
    w1 = w1_ref[0, 0]
    w3 = w3_ref[0, 0]
    h1 = jax.lax.dot_general(x, w1, (((1,), (1,)), ((), ())),
                             preferred_element_type=jnp.float32)
    h3 = jax.lax.dot_general(x, w3, (((1,), (1,)), ((), ())),
                             preferred_element_type=jnp.float32)
    act = h1 * jax.nn.sigmoid(h1) * h3
    w2 = w2_ref[0]
    contrib = jax.lax.dot_general(act, w2, (((1,), (1,)), ((), ())),
                                  preferred_element_type=jnp.float32)
    lanes = jax.lax.broadcasted_iota(jnp.int32, (_T, _E), 1)
    scale = jnp.sum(jnp.where(lanes == e, rt_ref[...], 0.0),
                    axis=-1, keepdims=True)
    out_ref[...] += scale * contrib


def kernel(hidden_states, gate_w, ws, w2s):
    ws4 = ws.reshape(_E, 2, _I, _H)
    grid = (_E, _NB)
    return pl.pallas_call(
        _moe_body,
        grid=grid,
        in_specs=[
            pl.BlockSpec((_T, _H), lambda e, i: (0, 0)),
            pl.BlockSpec((_E, _H), lambda e, i: (0, 0)),
            pl.BlockSpec((1, 1, _BI, _H), lambda e, i: (e, 0, i, 0)),
            pl.BlockSpec((1, 1, _BI, _H), lambda e, i: (e, 1, i, 0)),
            pl.BlockSpec((1, _H, _BI), lambda e, i: (e, 0, i)),
        ],
        out_specs=pl.BlockSpec((_T, _H), lambda e, i: (0, 0)),
        out_shape=jax.ShapeDtypeStruct((_T, _H), jnp.float32),
        scratch_shapes=[pltpu.VMEM((_T, _E), jnp.float32)],
    )(hidden_states, gate_w, ws4, ws4, w2s)
